# baseline (speedup 1.0000x reference)
import functools

import jax
import jax.numpy as jnp
from jax import lax
from jax.experimental import pallas as pl
from jax.experimental.pallas import tpu as pltpu
from jax.experimental.pallas import tpu_sc as plsc

_VOCAB = 100000
_MAX_LEN = 2048
_DIM = 1024
_BATCH = 4

_NC = 2
_NS = 16
_NW = _NC * _NS
_T_PER_W = _MAX_LEN // _NW
_CHUNK = 32
_LANES = 16

_mesh = plsc.VectorSubcoreMesh(core_axis_name="c", subcore_axis_name="s")


@functools.partial(
    pl.kernel,
    mesh=_mesh,
    out_type=jax.ShapeDtypeStruct((_BATCH * _MAX_LEN, _DIM), jnp.float32),
    scratch_types=[
        pltpu.VMEM((_CHUNK,), jnp.int32),
        pltpu.VMEM((_T_PER_W, _DIM), jnp.float32),
        pltpu.VMEM((_CHUNK, _DIM), jnp.float32),
        pltpu.SemaphoreType.DMA,
    ],
)
def _embed(idx_hbm, table_hbm, pos_hbm, out_hbm, idx_v, pos_v, rows_v, sem):
    wid = lax.axis_index("s") * _NC + lax.axis_index("c")
    t0 = wid * _T_PER_W
    pltpu.sync_copy(pos_hbm.at[pl.ds(t0, _T_PER_W)], pos_v)
    for b in range(_BATCH):
        for h in range(_T_PER_W // _CHUNK):
            base = b * _MAX_LEN + t0 + h * _CHUNK
            pltpu.sync_copy(idx_hbm.at[pl.ds(base, _CHUNK)], idx_v)
            pltpu.async_copy(table_hbm.at[idx_v], rows_v, sem).wait()

            def add_row(r, _, h=h):
                for c in range(_DIM // _LANES):
                    sl = pl.ds(c * _LANES, _LANES)
                    rows_v[r, sl] = rows_v[r, sl] + pos_v[h * _CHUNK + r, sl]
                return 0

            lax.fori_loop(0, _CHUNK, add_row, 0)
            pltpu.sync_copy(rows_v, out_hbm.at[pl.ds(base, _CHUNK)])


def kernel(inputs, token_table, pos_table):
    flat_idx = inputs.reshape(-1).astype(jnp.int32)
    out = _embed(flat_idx, token_table, pos_table)
    return out.reshape(_BATCH, _MAX_LEN, _DIM)


# speedup vs baseline: 1.1181x; 1.1181x over previous
import functools

import jax
import jax.numpy as jnp
from jax import lax
from jax.experimental import pallas as pl
from jax.experimental.pallas import tpu as pltpu
from jax.experimental.pallas import tpu_sc as plsc

_VOCAB = 100000
_MAX_LEN = 2048
_DIM = 1024
_BATCH = 4

_NC = 2
_NS = 16
_NW = _NC * _NS
_T_PER_W = _MAX_LEN // _NW
_CHUNK = 16
_NCHUNK = _BATCH * _T_PER_W // _CHUNK
_CPB = _T_PER_W // _CHUNK
_LANES = 16

_mesh = plsc.VectorSubcoreMesh(core_axis_name="c", subcore_axis_name="s")


@functools.partial(
    pl.kernel,
    mesh=_mesh,
    out_type=jax.ShapeDtypeStruct((_BATCH * _MAX_LEN, _DIM), jnp.float32),
    scratch_types=[
        pltpu.VMEM((_BATCH * _T_PER_W,), jnp.int32),
        pltpu.VMEM((_T_PER_W, _DIM), jnp.float32),
        pltpu.VMEM((_CHUNK, _DIM), jnp.float32),
        pltpu.VMEM((_CHUNK, _DIM), jnp.float32),
        pltpu.VMEM((_CHUNK, _DIM), jnp.float32),
        pltpu.SemaphoreType.DMA,
        pltpu.SemaphoreType.DMA,
        pltpu.SemaphoreType.DMA,
        pltpu.SemaphoreType.DMA,
        pltpu.SemaphoreType.DMA,
        pltpu.SemaphoreType.DMA,
    ],
)
def _embed(idx_hbm, table_hbm, pos_hbm, out_hbm,
           idx_v, pos_v, rows0, rows1, rows2, g0, g1, g2, w0, w1, w2):
    wid = lax.axis_index("s") * _NC + lax.axis_index("c")
    t0 = wid * _T_PER_W
    bufs = (rows0, rows1, rows2)
    gsems = (g0, g1, g2)
    wsems = (w0, w1, w2)
    nbuf = len(bufs)

    for b in range(_BATCH):
        pltpu.sync_copy(idx_hbm.at[pl.ds(b * _MAX_LEN + t0, _T_PER_W)],
                        idx_v.at[pl.ds(b * _T_PER_W, _T_PER_W)])
    pltpu.sync_copy(pos_hbm.at[pl.ds(t0, _T_PER_W)], pos_v)

    def out_base(c):
        b, h = divmod(c, _CPB)
        return b * _MAX_LEN + t0 + h * _CHUNK

    def gather(c):
        return pltpu.async_copy(
            table_hbm.at[idx_v.at[pl.ds(c * _CHUNK, _CHUNK)]],
            bufs[c % nbuf], gsems[c % nbuf])

    def writeback(c):
        return pltpu.async_copy(
            bufs[c % nbuf], out_hbm.at[pl.ds(out_base(c), _CHUNK)],
            wsems[c % nbuf])

    hw = [None] * nbuf
    hg = gather(0)
    for c in range(_NCHUNK):
        hg.wait()
        if c + 1 < _NCHUNK:
            nxt = (c + 1) % nbuf
            if hw[nxt] is not None:
                hw[nxt].wait()
                hw[nxt] = None
            hg = gather(c + 1)
        buf = bufs[c % nbuf]
        prow = (c % _CPB) * _CHUNK

        def add_row(r, _, buf=buf, prow=prow):
            for cc in range(_DIM // _LANES):
                sl = pl.ds(cc * _LANES, _LANES)
                buf[r, sl] = buf[r, sl] + pos_v[prow + r, sl]
            return 0

        lax.fori_loop(0, _CHUNK, add_row, 0)
        hw[c % nbuf] = writeback(c)
    for h in hw:
        if h is not None:
            h.wait()


def kernel(inputs, token_table, pos_table):
    flat_idx = inputs.reshape(-1).astype(jnp.int32)
    out = _embed(flat_idx, token_table, pos_table)
    return out.reshape(_BATCH, _MAX_LEN, _DIM)


# speedup vs baseline: 1.2725x; 1.1381x over previous
import functools

import jax
import jax.numpy as jnp
from jax import lax
from jax.experimental import pallas as pl
from jax.experimental.pallas import tpu as pltpu
from jax.experimental.pallas import tpu_sc as plsc

_VOCAB = 100000
_MAX_LEN = 2048
_DIM = 1024
_BATCH = 4

_NC = 2
_NS = 16
_NW = _NC * _NS
_T_PER_W = _MAX_LEN // _NW
_CHUNK = 16
_NCHUNK = _BATCH * _T_PER_W // _CHUNK
_CPB = _T_PER_W // _CHUNK
_LANES = 16

_mesh = plsc.VectorSubcoreMesh(core_axis_name="c", subcore_axis_name="s")


@functools.partial(
    pl.kernel,
    mesh=_mesh,
    out_type=jax.ShapeDtypeStruct((_BATCH * _MAX_LEN, _DIM), jnp.float32),
    scratch_types=[
        pltpu.VMEM((_BATCH * _T_PER_W,), jnp.int32),
        pltpu.VMEM((_T_PER_W, _DIM), jnp.float32),
        pltpu.VMEM((_CHUNK, _DIM), jnp.float32),
        pltpu.VMEM((_CHUNK, _DIM), jnp.float32),
        pltpu.VMEM((_CHUNK, _DIM), jnp.float32),
        pltpu.SemaphoreType.DMA,
        pltpu.SemaphoreType.DMA,
        pltpu.SemaphoreType.DMA,
        pltpu.SemaphoreType.DMA,
        pltpu.SemaphoreType.DMA,
        pltpu.SemaphoreType.DMA,
    ],
)
def _embed(idx_hbm, table_hbm, pos_hbm, out_hbm,
           idx_v, pos_v, rows0, rows1, rows2, g0, g1, g2, w0, w1, w2):
    wid = lax.axis_index("s") * _NC + lax.axis_index("c")
    t0 = wid * _T_PER_W
    bufs = (rows0, rows1, rows2)
    gsems = (g0, g1, g2)
    wsems = (w0, w1, w2)
    nbuf = len(bufs)

    for b in range(_BATCH):
        pltpu.sync_copy(idx_hbm.at[pl.ds(b * _MAX_LEN + t0, _T_PER_W)],
                        idx_v.at[pl.ds(b * _T_PER_W, _T_PER_W)])
    pltpu.sync_copy(pos_hbm.at[pl.ds(t0, _T_PER_W)], pos_v)

    def out_base(c):
        b, h = divmod(c, _CPB)
        return b * _MAX_LEN + t0 + h * _CHUNK

    def gather(c):
        return pltpu.async_copy(
            table_hbm.at[idx_v.at[pl.ds(c * _CHUNK, _CHUNK)]],
            bufs[c % nbuf], gsems[c % nbuf])

    def writeback(c):
        return pltpu.async_copy(
            bufs[c % nbuf], out_hbm.at[pl.ds(out_base(c), _CHUNK)],
            wsems[c % nbuf])

    hw = [None] * nbuf
    hg = gather(0)
    for c in range(_NCHUNK):
        hg.wait()
        if c + 1 < _NCHUNK:
            nxt = (c + 1) % nbuf
            if hw[nxt] is not None:
                hw[nxt].wait()
                hw[nxt] = None
            hg = gather(c + 1)
        buf = bufs[c % nbuf]
        prow = (c % _CPB) * _CHUNK

        def add_row(r, _, buf=buf, prow=prow):
            for cc in range(_DIM // _LANES):
                sl = pl.ds(cc * _LANES, _LANES)
                plsc.addupdate(buf.at[r, sl], pos_v[prow + r, sl])
            return 0

        lax.fori_loop(0, _CHUNK, add_row, 0)
        hw[c % nbuf] = writeback(c)
    for h in hw:
        if h is not None:
            h.wait()


def kernel(inputs, token_table, pos_table):
    flat_idx = inputs.reshape(-1).astype(jnp.int32)
    out = _embed(flat_idx, token_table, pos_table)
    return out.reshape(_BATCH, _MAX_LEN, _DIM)


# speedup vs baseline: 1.5368x; 1.2077x over previous
import functools

import jax
import jax.numpy as jnp
from jax import lax
from jax.experimental import pallas as pl
from jax.experimental.pallas import tpu as pltpu
from jax.experimental.pallas import tpu_sc as plsc

_VOCAB = 100000
_MAX_LEN = 2048
_DIM = 1024
_BATCH = 4

_NC = 2
_NS = 16
_NW = _NC * _NS
_T_PER_W = _MAX_LEN // _NW
_CHUNK = 16
_NCHUNK = _BATCH * _T_PER_W // _CHUNK
_NH = _T_PER_W // _CHUNK
_LANES = 16
_NBUF = 5
_GDEPTH = 3

_mesh = plsc.VectorSubcoreMesh(core_axis_name="c", subcore_axis_name="s")


@functools.partial(
    pl.kernel,
    mesh=_mesh,
    out_type=jax.ShapeDtypeStruct((_BATCH, _MAX_LEN, _DIM), jnp.float32),
    scratch_types=(
        [pltpu.VMEM((_BATCH * _T_PER_W,), jnp.int32)]
        + [pltpu.VMEM((_CHUNK, _DIM), jnp.float32) for _ in range(2)]
        + [pltpu.VMEM((_CHUNK, _DIM), jnp.float32) for _ in range(_NBUF)]
        + [pltpu.SemaphoreType.DMA for _ in range(2 + 2 * _NBUF)]
    ),
)
def _embed(idx_hbm, table_hbm, pos_hbm, out_hbm, idx_v, *scratch):
    pring = scratch[:2]
    bufs = scratch[2:2 + _NBUF]
    psems = scratch[2 + _NBUF:4 + _NBUF]
    gsems = scratch[4 + _NBUF:4 + 2 * _NBUF]
    wsems = scratch[4 + 2 * _NBUF:4 + 3 * _NBUF]

    wid = lax.axis_index("s") * _NC + lax.axis_index("c")
    t0 = wid * _T_PER_W

    for b in range(_BATCH):
        pltpu.sync_copy(idx_hbm.at[b, pl.ds(t0, _T_PER_W)],
                        idx_v.at[pl.ds(b * _T_PER_W, _T_PER_W)])

    def pos_load(h):
        return pltpu.async_copy(
            pos_hbm.at[pl.ds(t0 + h * _CHUNK, _CHUNK)],
            pring[h % 2], psems[h % 2])

    def gather(c):
        h, b = divmod(c, _BATCH)
        return pltpu.async_copy(
            table_hbm.at[idx_v.at[pl.ds(b * _T_PER_W + h * _CHUNK, _CHUNK)]],
            bufs[c % _NBUF], gsems[c % _NBUF])

    def writeback(c):
        h, b = divmod(c, _BATCH)
        return pltpu.async_copy(
            bufs[c % _NBUF],
            out_hbm.at[b, pl.ds(t0 + h * _CHUNK, _CHUNK)],
            wsems[c % _NBUF])

    hp = [pos_load(0), pos_load(1)]
    pos_ready = [False, False]
    hw = [None] * _NBUF
    hg = [None] * _NBUF
    issued = 0
    for c in range(_NCHUNK):
        h = c // _BATCH
        while issued < min(c + 1 + _GDEPTH, _NCHUNK):
            slot = issued % _NBUF
            if hw[slot] is not None:
                hw[slot].wait()
                hw[slot] = None
            hg[slot] = gather(issued)
            issued += 1
        hg[c % _NBUF].wait()
        if not pos_ready[h % 2]:
            hp[h % 2].wait()
            pos_ready[h % 2] = True
        buf = bufs[c % _NBUF]
        pos = pring[h % 2]

        def add_row(r, _, buf=buf, pos=pos):
            for cc in range(_DIM // _LANES):
                sl = pl.ds(cc * _LANES, _LANES)
                plsc.addupdate(buf.at[r, sl], pos[r, sl])
            return 0

        lax.fori_loop(0, _CHUNK, add_row, 0)
        if c % _BATCH == _BATCH - 1:
            pos_ready[h % 2] = False
            if h + 2 <= _NH - 1:
                hp[h % 2] = pos_load(h + 2)
        hw[c % _NBUF] = writeback(c)
    for hnd in hw:
        if hnd is not None:
            hnd.wait()


def kernel(inputs, token_table, pos_table):
    return _embed(inputs.astype(jnp.int32), token_table, pos_table)


# speedup vs baseline: 1.5684x; 1.0205x over previous
import functools

import jax
import jax.numpy as jnp
from jax import lax
from jax.experimental import pallas as pl
from jax.experimental.pallas import tpu as pltpu
from jax.experimental.pallas import tpu_sc as plsc

_VOCAB = 100000
_MAX_LEN = 2048
_DIM = 1024
_BATCH = 4

_NC = 2
_NS = 16
_NW = _NC * _NS
_T_PER_W = _MAX_LEN // _NW
_CHUNK = 16
_NCHUNK = _BATCH * _T_PER_W // _CHUNK
_NH = _T_PER_W // _CHUNK
_LANES = 16
_NBUF = 5
_GDEPTH = 3

_mesh = plsc.VectorSubcoreMesh(core_axis_name="c", subcore_axis_name="s")


@functools.partial(
    pl.kernel,
    mesh=_mesh,
    out_type=jax.ShapeDtypeStruct((_BATCH, _MAX_LEN, _DIM), jnp.float32),
    scratch_types=(
        [pltpu.VMEM((_BATCH * _T_PER_W,), jnp.int32)]
        + [pltpu.VMEM((_CHUNK, _DIM), jnp.float32) for _ in range(2)]
        + [pltpu.VMEM((_CHUNK, _DIM), jnp.float32) for _ in range(_NBUF)]
        + [pltpu.SemaphoreType.DMA for _ in range(3 + 2 * _NBUF)]
    ),
)
def _embed(idx_hbm, table_hbm, pos_hbm, out_hbm, idx_v, *scratch):
    pring = scratch[:2]
    bufs = scratch[2:2 + _NBUF]
    psems = scratch[2 + _NBUF:4 + _NBUF]
    isem = scratch[4 + _NBUF]
    gsems = scratch[5 + _NBUF:5 + 2 * _NBUF]
    wsems = scratch[5 + 2 * _NBUF:5 + 3 * _NBUF]

    wid = lax.axis_index("s") * _NC + lax.axis_index("c")
    t0 = wid * _T_PER_W

    idx_handles = [
        pltpu.async_copy(idx_hbm.at[b, pl.ds(t0, _T_PER_W)],
                         idx_v.at[pl.ds(b * _T_PER_W, _T_PER_W)], isem)
        for b in range(_BATCH)
    ]

    def pos_load(h):
        return pltpu.async_copy(
            pos_hbm.at[pl.ds(t0 + h * _CHUNK, _CHUNK)],
            pring[h % 2], psems[h % 2])

    def gather(c):
        h, b = divmod(c, _BATCH)
        return pltpu.async_copy(
            table_hbm.at[idx_v.at[pl.ds(b * _T_PER_W + h * _CHUNK, _CHUNK)]],
            bufs[c % _NBUF], gsems[c % _NBUF])

    def writeback(c):
        h, b = divmod(c, _BATCH)
        return pltpu.async_copy(
            bufs[c % _NBUF],
            out_hbm.at[b, pl.ds(t0 + h * _CHUNK, _CHUNK)],
            wsems[c % _NBUF])

    hp = [pos_load(0), pos_load(1)]
    for hnd in idx_handles:
        hnd.wait()
    pos_ready = [False, False]
    hw = [None] * _NBUF
    hg = [None] * _NBUF
    issued = 0
    for c in range(_NCHUNK):
        h = c // _BATCH
        while issued < min(c + 1 + _GDEPTH, _NCHUNK):
            slot = issued % _NBUF
            if hw[slot] is not None:
                hw[slot].wait()
                hw[slot] = None
            hg[slot] = gather(issued)
            issued += 1
        hg[c % _NBUF].wait()
        if not pos_ready[h % 2]:
            hp[h % 2].wait()
            pos_ready[h % 2] = True
        buf = bufs[c % _NBUF]
        pos = pring[h % 2]

        def add_row(r, _, buf=buf, pos=pos):
            for cc in range(_DIM // _LANES):
                sl = pl.ds(cc * _LANES, _LANES)
                plsc.addupdate(buf.at[r, sl], pos[r, sl])
            return 0

        lax.fori_loop(0, _CHUNK, add_row, 0)
        if c % _BATCH == _BATCH - 1:
            pos_ready[h % 2] = False
            if h + 2 <= _NH - 1:
                hp[h % 2] = pos_load(h + 2)
        hw[c % _NBUF] = writeback(c)
    for hnd in hw:
        if hnd is not None:
            hnd.wait()


def kernel(inputs, token_table, pos_table):
    return _embed(inputs.astype(jnp.int32), token_table, pos_table)
